# baseline (device time: 390710 ns/iter reference)
import jax
import jax.numpy as jnp
from jax import lax
from jax.experimental import pallas as pl
from jax.experimental.pallas import tpu as pltpu

N_DEV = 4
M = 4096
K_SHARD = 1024
N_GLOBAL = 8192
M_CHUNK = M // N_DEV
TILE_N = 1024
STEP_N = 2 * TILE_N
N_STEPS = N_GLOBAL // STEP_N


def kernel(x, w_mat):
    def body(x_ref, w_ref, o_ref, comm_a, comm_b, stage,
             send_a, recv_a, send_b, recv_b, copy_sems):
        t = pl.program_id(0)
        d = lax.axis_index("i")
        left = lax.rem(d + N_DEV - 1, N_DEV)
        right = lax.rem(d + 1, N_DEV)
        col_a = t * STEP_N
        col_b = col_a + TILE_N

        def out_copy(k, col):
            return pltpu.make_async_copy(
                stage.at[k],
                o_ref.at[:, pl.ds(col, TILE_N)],
                copy_sems.at[k],
            )

        barrier = pltpu.get_barrier_semaphore()
        pl.semaphore_signal(
            barrier, inc=1, device_id=(left,),
            device_id_type=pl.DeviceIdType.MESH,
        )
        pl.semaphore_signal(
            barrier, inc=1, device_id=(right,),
            device_id_type=pl.DeviceIdType.MESH,
        )
        pl.semaphore_wait(barrier, 2)

        @pl.when(t > 0)
        def _():
            out_copy(0, col_a - STEP_N).wait()
            out_copy(1, col_b - STEP_N).wait()

        def part(c, col):
            return jnp.dot(
                x_ref[pl.ds(c * M_CHUNK, M_CHUNK), :],
                w_ref[:, pl.ds(col, TILE_N)],
                preferred_element_type=jnp.float32,
            )

        @pl.when(t == 0)
        def _():
            comm_a[0, :, :] = part(lax.rem(d + N_DEV - 1, N_DEV),
                                   col_a).astype(jnp.bfloat16)
            comm_b[0, :, :] = part(lax.rem(d + 1, N_DEV),
                                   col_b).astype(jnp.bfloat16)

        for s in range(N_DEV - 1):
            rdma_a = pltpu.make_async_remote_copy(
                src_ref=comm_a.at[s],
                dst_ref=comm_a.at[s + 1],
                send_sem=send_a.at[s],
                recv_sem=recv_a.at[s],
                device_id=(right,),
                device_id_type=pl.DeviceIdType.MESH,
            )
            rdma_a.start()
            rdma_b = pltpu.make_async_remote_copy(
                src_ref=comm_b.at[s],
                dst_ref=comm_b.at[s + 1],
                send_sem=send_b.at[s],
                recv_sem=recv_b.at[s],
                device_id=(left,),
                device_id_type=pl.DeviceIdType.MESH,
            )
            rdma_b.start()
            ca = lax.rem(d + 2 * N_DEV - 2 - s, N_DEV)
            cb = lax.rem(d + 2 + s, N_DEV)
            pa = part(ca, col_a).astype(jnp.bfloat16)
            pb = part(cb, col_b).astype(jnp.bfloat16)
            if s == 1:
                @pl.when(t + 1 < N_STEPS)
                def _():
                    comm_a[0, :, :] = part(
                        lax.rem(d + N_DEV - 1, N_DEV),
                        col_a + STEP_N).astype(jnp.bfloat16)
                    comm_b[0, :, :] = part(
                        lax.rem(d + 1, N_DEV),
                        col_b + STEP_N).astype(jnp.bfloat16)
            rdma_a.wait()
            comm_a[s + 1, :, :] = (
                comm_a[s + 1, :, :].astype(jnp.float32)
                + pa.astype(jnp.float32)
            ).astype(jnp.bfloat16)
            rdma_b.wait()
            comm_b[s + 1, :, :] = (
                comm_b[s + 1, :, :].astype(jnp.float32)
                + pb.astype(jnp.float32)
            ).astype(jnp.bfloat16)

        ya = comm_a[N_DEV - 1, :, :].astype(jnp.float32)
        yb = comm_b[N_DEV - 1, :, :].astype(jnp.float32)
        stage[0, :, :] = ya * jax.nn.sigmoid(ya)
        stage[1, :, :] = yb * jax.nn.sigmoid(yb)
        out_copy(0, col_a).start()
        out_copy(1, col_b).start()

        @pl.when(t == N_STEPS - 1)
        def _():
            out_copy(0, col_a).wait()
            out_copy(1, col_b).wait()

    return pl.pallas_call(
        body,
        grid=(N_STEPS,),
        in_specs=[
            pl.BlockSpec((M, K_SHARD), lambda t: (0, 0)),
            pl.BlockSpec((K_SHARD, N_GLOBAL), lambda t: (0, 0)),
        ],
        out_specs=pl.BlockSpec(memory_space=pl.ANY),
        out_shape=jax.ShapeDtypeStruct((M_CHUNK, N_GLOBAL), jnp.float32),
        scratch_shapes=[
            pltpu.VMEM((N_DEV, M_CHUNK, TILE_N), jnp.bfloat16),
            pltpu.VMEM((N_DEV, M_CHUNK, TILE_N), jnp.bfloat16),
            pltpu.VMEM((2, M_CHUNK, TILE_N), jnp.float32),
            pltpu.SemaphoreType.DMA((N_DEV - 1,)),
            pltpu.SemaphoreType.DMA((N_DEV - 1,)),
            pltpu.SemaphoreType.DMA((N_DEV - 1,)),
            pltpu.SemaphoreType.DMA((N_DEV - 1,)),
            pltpu.SemaphoreType.DMA((2,)),
        ],
        compiler_params=pltpu.CompilerParams(
            collective_id=0,
            dimension_semantics=("arbitrary",),
            vmem_limit_bytes=60 * 1024 * 1024,
        ),
    )(x.astype(jnp.bfloat16), w_mat.astype(jnp.bfloat16))


# device time: 373994 ns/iter; 1.0447x vs baseline; 1.0447x over previous
import jax
import jax.numpy as jnp
from jax import lax
from jax.experimental import pallas as pl
from jax.experimental.pallas import tpu as pltpu

N_DEV = 4
M = 4096
K_SHARD = 1024
N_GLOBAL = 8192
M_CHUNK = M // N_DEV
TILE_N = 1024
STEP_N = 2 * TILE_N
N_STEPS = N_GLOBAL // STEP_N


def kernel(x, w_mat):
    def body(x_ref, w_ref, o_ref, comm_a, comm_b,
             send_a, recv_a, send_b, recv_b):
        t = pl.program_id(0)
        d = lax.axis_index("i")
        left = lax.rem(d + N_DEV - 1, N_DEV)
        right = lax.rem(d + 1, N_DEV)
        col_a = t * STEP_N
        col_b = col_a + TILE_N

        barrier = pltpu.get_barrier_semaphore()
        pl.semaphore_signal(
            barrier, inc=1, device_id=(left,),
            device_id_type=pl.DeviceIdType.MESH,
        )
        pl.semaphore_signal(
            barrier, inc=1, device_id=(right,),
            device_id_type=pl.DeviceIdType.MESH,
        )
        pl.semaphore_wait(barrier, 2)

        def part(c, col):
            return jnp.dot(
                x_ref[pl.ds(c * M_CHUNK, M_CHUNK), :],
                w_ref[:, pl.ds(col, TILE_N)],
                preferred_element_type=jnp.float32,
            )

        @pl.when(t == 0)
        def _():
            comm_a[0, :, :] = part(lax.rem(d + N_DEV - 1, N_DEV),
                                   col_a).astype(jnp.bfloat16)
            comm_b[0, :, :] = part(lax.rem(d + 1, N_DEV),
                                   col_b).astype(jnp.bfloat16)

        for s in range(N_DEV - 1):
            rdma_a = pltpu.make_async_remote_copy(
                src_ref=comm_a.at[s],
                dst_ref=comm_a.at[s + 1],
                send_sem=send_a.at[s],
                recv_sem=recv_a.at[s],
                device_id=(right,),
                device_id_type=pl.DeviceIdType.MESH,
            )
            rdma_a.start()
            rdma_b = pltpu.make_async_remote_copy(
                src_ref=comm_b.at[s],
                dst_ref=comm_b.at[s + 1],
                send_sem=send_b.at[s],
                recv_sem=recv_b.at[s],
                device_id=(left,),
                device_id_type=pl.DeviceIdType.MESH,
            )
            rdma_b.start()
            ca = lax.rem(d + 2 * N_DEV - 2 - s, N_DEV)
            cb = lax.rem(d + 2 + s, N_DEV)
            pa = part(ca, col_a).astype(jnp.bfloat16)
            pb = part(cb, col_b).astype(jnp.bfloat16)
            if s == 1:
                @pl.when(t + 1 < N_STEPS)
                def _():
                    comm_a[0, :, :] = part(
                        lax.rem(d + N_DEV - 1, N_DEV),
                        col_a + STEP_N).astype(jnp.bfloat16)
                    comm_b[0, :, :] = part(
                        lax.rem(d + 1, N_DEV),
                        col_b + STEP_N).astype(jnp.bfloat16)
            rdma_a.wait()
            comm_a[s + 1, :, :] = (
                comm_a[s + 1, :, :].astype(jnp.float32)
                + pa.astype(jnp.float32)
            ).astype(jnp.bfloat16)
            rdma_b.wait()
            comm_b[s + 1, :, :] = (
                comm_b[s + 1, :, :].astype(jnp.float32)
                + pb.astype(jnp.float32)
            ).astype(jnp.bfloat16)

        ya = comm_a[N_DEV - 1, :, :].astype(jnp.float32)
        yb = comm_b[N_DEV - 1, :, :].astype(jnp.float32)
        o_ref[:, :TILE_N] = (ya * jax.nn.sigmoid(ya)).astype(jnp.bfloat16)
        o_ref[:, TILE_N:] = (yb * jax.nn.sigmoid(yb)).astype(jnp.bfloat16)

    return pl.pallas_call(
        body,
        grid=(N_STEPS,),
        in_specs=[
            pl.BlockSpec((M, K_SHARD), lambda t: (0, 0)),
            pl.BlockSpec((K_SHARD, N_GLOBAL), lambda t: (0, 0)),
        ],
        out_specs=pl.BlockSpec((M_CHUNK, STEP_N), lambda t: (0, t)),
        out_shape=jax.ShapeDtypeStruct((M_CHUNK, N_GLOBAL), jnp.bfloat16),
        scratch_shapes=[
            pltpu.VMEM((N_DEV, M_CHUNK, TILE_N), jnp.bfloat16),
            pltpu.VMEM((N_DEV, M_CHUNK, TILE_N), jnp.bfloat16),
            pltpu.SemaphoreType.DMA((N_DEV - 1,)),
            pltpu.SemaphoreType.DMA((N_DEV - 1,)),
            pltpu.SemaphoreType.DMA((N_DEV - 1,)),
            pltpu.SemaphoreType.DMA((N_DEV - 1,)),
        ],
        compiler_params=pltpu.CompilerParams(
            collective_id=0,
            dimension_semantics=("arbitrary",),
            vmem_limit_bytes=60 * 1024 * 1024,
        ),
    )(x.astype(jnp.bfloat16), w_mat.astype(jnp.bfloat16))


# device time: 367088 ns/iter; 1.0643x vs baseline; 1.0188x over previous
import jax
import jax.numpy as jnp
from jax import lax
from jax.experimental import pallas as pl
from jax.experimental.pallas import tpu as pltpu

N_DEV = 4
M = 4096
K_SHARD = 1024
N_GLOBAL = 8192
M_CHUNK = M // N_DEV
TILE_N = 1024
STEP_N = 2 * TILE_N
N_STEPS = N_GLOBAL // STEP_N


def kernel(x, w_mat):
    def body(x_ref, w_ref, o_ref, comm_a, comm_b,
             send_a, recv_a, send_b, recv_b):
        t = pl.program_id(0)
        d = lax.axis_index("i")
        left = lax.rem(d + N_DEV - 1, N_DEV)
        right = lax.rem(d + 1, N_DEV)
        col_a = t * STEP_N
        col_b = col_a + TILE_N

        barrier = pltpu.get_barrier_semaphore()
        pl.semaphore_signal(
            barrier, inc=1, device_id=(left,),
            device_id_type=pl.DeviceIdType.MESH,
        )
        pl.semaphore_signal(
            barrier, inc=1, device_id=(right,),
            device_id_type=pl.DeviceIdType.MESH,
        )
        pl.semaphore_wait(barrier, 2)

        def part(c, col):
            return jnp.dot(
                x_ref[pl.ds(c * M_CHUNK, M_CHUNK), :],
                w_ref[:, pl.ds(col, TILE_N)],
                preferred_element_type=jnp.float32,
            )

        @pl.when(t == 0)
        def _():
            comm_a[0, :, :] = part(lax.rem(d + N_DEV - 1, N_DEV),
                                   col_a).astype(jnp.bfloat16)
            comm_b[0, :, :] = part(lax.rem(d + 1, N_DEV),
                                   col_b).astype(jnp.bfloat16)

        HALF = M_CHUNK // 2

        def hop_rdma(comm, sems_s, sems_r, s, h, nbr):
            lo = h * HALF
            return pltpu.make_async_remote_copy(
                src_ref=comm.at[s, lo:lo + HALF],
                dst_ref=comm.at[s + 1, lo:lo + HALF],
                send_sem=sems_s.at[s, h],
                recv_sem=sems_r.at[s, h],
                device_id=(nbr,),
                device_id_type=pl.DeviceIdType.MESH,
            )

        for s in range(N_DEV - 1):
            rdmas = []
            for h in range(2):
                ra = hop_rdma(comm_a, send_a, recv_a, s, h, right)
                rb = hop_rdma(comm_b, send_b, recv_b, s, h, left)
                ra.start()
                rb.start()
                rdmas.append((ra, rb))
            ca = lax.rem(d + 2 * N_DEV - 2 - s, N_DEV)
            cb = lax.rem(d + 2 + s, N_DEV)
            pa = part(ca, col_a).astype(jnp.bfloat16)
            pb = part(cb, col_b).astype(jnp.bfloat16)
            if s == 1:
                @pl.when(t + 1 < N_STEPS)
                def _():
                    comm_a[0, :, :] = part(
                        lax.rem(d + N_DEV - 1, N_DEV),
                        col_a + STEP_N).astype(jnp.bfloat16)
                    comm_b[0, :, :] = part(
                        lax.rem(d + 1, N_DEV),
                        col_b + STEP_N).astype(jnp.bfloat16)
            for h in range(2):
                ra, rb = rdmas[h]
                lo = h * HALF
                ra.wait()
                comm_a[s + 1, lo:lo + HALF, :] = (
                    comm_a[s + 1, lo:lo + HALF, :].astype(jnp.float32)
                    + pa[lo:lo + HALF, :].astype(jnp.float32)
                ).astype(jnp.bfloat16)
                rb.wait()
                comm_b[s + 1, lo:lo + HALF, :] = (
                    comm_b[s + 1, lo:lo + HALF, :].astype(jnp.float32)
                    + pb[lo:lo + HALF, :].astype(jnp.float32)
                ).astype(jnp.bfloat16)

        ya = comm_a[N_DEV - 1, :, :].astype(jnp.float32)
        yb = comm_b[N_DEV - 1, :, :].astype(jnp.float32)
        o_ref[:, :TILE_N] = (ya * jax.nn.sigmoid(ya)).astype(jnp.bfloat16)
        o_ref[:, TILE_N:] = (yb * jax.nn.sigmoid(yb)).astype(jnp.bfloat16)

    return pl.pallas_call(
        body,
        grid=(N_STEPS,),
        in_specs=[
            pl.BlockSpec((M, K_SHARD), lambda t: (0, 0)),
            pl.BlockSpec((K_SHARD, N_GLOBAL), lambda t: (0, 0)),
        ],
        out_specs=pl.BlockSpec((M_CHUNK, STEP_N), lambda t: (0, t)),
        out_shape=jax.ShapeDtypeStruct((M_CHUNK, N_GLOBAL), jnp.bfloat16),
        scratch_shapes=[
            pltpu.VMEM((N_DEV, M_CHUNK, TILE_N), jnp.bfloat16),
            pltpu.VMEM((N_DEV, M_CHUNK, TILE_N), jnp.bfloat16),
            pltpu.SemaphoreType.DMA((N_DEV - 1, 2)),
            pltpu.SemaphoreType.DMA((N_DEV - 1, 2)),
            pltpu.SemaphoreType.DMA((N_DEV - 1, 2)),
            pltpu.SemaphoreType.DMA((N_DEV - 1, 2)),
        ],
        compiler_params=pltpu.CompilerParams(
            collective_id=0,
            dimension_semantics=("arbitrary",),
            vmem_limit_bytes=60 * 1024 * 1024,
        ),
    )(x.astype(jnp.bfloat16), w_mat.astype(jnp.bfloat16))


# device time: 348359 ns/iter; 1.1216x vs baseline; 1.0538x over previous
import jax
import jax.numpy as jnp
from jax import lax
from jax.experimental import pallas as pl
from jax.experimental.pallas import tpu as pltpu

N_DEV = 4
M = 4096
K_SHARD = 1024
N_GLOBAL = 8192
M_CHUNK = M // N_DEV
TILE_N = 1024
STEP_N = 2 * TILE_N
N_STEPS = N_GLOBAL // STEP_N


def kernel(x, w_mat):
    def body(x_ref, w_ref, o_ref, comm_a, comm_b,
             send_a, recv_a, send_b, recv_b):
        t = pl.program_id(0)
        d = lax.axis_index("i")
        left = lax.rem(d + N_DEV - 1, N_DEV)
        right = lax.rem(d + 1, N_DEV)
        col_a = t * STEP_N
        col_b = col_a + TILE_N

        barrier = pltpu.get_barrier_semaphore()
        pl.semaphore_signal(
            barrier, inc=1, device_id=(left,),
            device_id_type=pl.DeviceIdType.MESH,
        )
        pl.semaphore_signal(
            barrier, inc=1, device_id=(right,),
            device_id_type=pl.DeviceIdType.MESH,
        )
        pl.semaphore_wait(barrier, 2)

        def part(c, col):
            return jnp.dot(
                x_ref[pl.ds(c * M_CHUNK, M_CHUNK), :],
                w_ref[:, pl.ds(col, TILE_N)],
                preferred_element_type=jnp.float32,
            )

        @pl.when(t == 0)
        def _():
            comm_a[0, :, :] = part(lax.rem(d + N_DEV - 1, N_DEV),
                                   col_a).astype(jnp.bfloat16)
            comm_b[0, :, :] = part(lax.rem(d + 1, N_DEV),
                                   col_b).astype(jnp.bfloat16)

        HALF = M_CHUNK // 2

        def hop_rdma(comm, sems_s, sems_r, s, h, nbr):
            lo = h * HALF
            return pltpu.make_async_remote_copy(
                src_ref=comm.at[s, lo:lo + HALF],
                dst_ref=comm.at[s + 1, lo:lo + HALF],
                send_sem=sems_s.at[s, h],
                recv_sem=sems_r.at[s, h],
                device_id=(nbr,),
                device_id_type=pl.DeviceIdType.MESH,
            )

        def add_half(comm, p, s, h):
            lo = h * HALF
            comm[s + 1, lo:lo + HALF, :] = (
                comm[s + 1, lo:lo + HALF, :].astype(jnp.float32)
                + p[lo:lo + HALF, :].astype(jnp.float32)
            ).astype(jnp.bfloat16)

        hop_rdma(comm_a, send_a, recv_a, 0, 0, right).start()
        hop_rdma(comm_b, send_b, recv_b, 0, 0, left).start()
        hop_rdma(comm_a, send_a, recv_a, 0, 1, right).start()
        hop_rdma(comm_b, send_b, recv_b, 0, 1, left).start()
        pa = part(lax.rem(d + 2 * N_DEV - 2, N_DEV),
                  col_a).astype(jnp.bfloat16)
        pb = part(lax.rem(d + 2, N_DEV), col_b).astype(jnp.bfloat16)
        for s in range(N_DEV - 1):
            hop_rdma(comm_a, send_a, recv_a, s, 0, right).wait()
            add_half(comm_a, pa, s, 0)
            if s < N_DEV - 2:
                hop_rdma(comm_a, send_a, recv_a, s + 1, 0, right).start()
            hop_rdma(comm_b, send_b, recv_b, s, 0, left).wait()
            add_half(comm_b, pb, s, 0)
            if s < N_DEV - 2:
                hop_rdma(comm_b, send_b, recv_b, s + 1, 0, left).start()
            hop_rdma(comm_a, send_a, recv_a, s, 1, right).wait()
            add_half(comm_a, pa, s, 1)
            if s < N_DEV - 2:
                hop_rdma(comm_a, send_a, recv_a, s + 1, 1, right).start()
            hop_rdma(comm_b, send_b, recv_b, s, 1, left).wait()
            add_half(comm_b, pb, s, 1)
            if s < N_DEV - 2:
                hop_rdma(comm_b, send_b, recv_b, s + 1, 1, left).start()
                ca = lax.rem(d + 2 * N_DEV - 3 - s, N_DEV)
                cb = lax.rem(d + 3 + s, N_DEV)
                pa = part(ca, col_a).astype(jnp.bfloat16)
                pb = part(cb, col_b).astype(jnp.bfloat16)
            if s == 1:
                @pl.when(t + 1 < N_STEPS)
                def _():
                    comm_a[0, :, :] = part(
                        lax.rem(d + N_DEV - 1, N_DEV),
                        col_a + STEP_N).astype(jnp.bfloat16)
                    comm_b[0, :, :] = part(
                        lax.rem(d + 1, N_DEV),
                        col_b + STEP_N).astype(jnp.bfloat16)

        ya = comm_a[N_DEV - 1, :, :].astype(jnp.float32)
        yb = comm_b[N_DEV - 1, :, :].astype(jnp.float32)
        o_ref[:, :TILE_N] = (ya * jax.nn.sigmoid(ya)).astype(jnp.bfloat16)
        o_ref[:, TILE_N:] = (yb * jax.nn.sigmoid(yb)).astype(jnp.bfloat16)

    return pl.pallas_call(
        body,
        grid=(N_STEPS,),
        in_specs=[
            pl.BlockSpec((M, K_SHARD), lambda t: (0, 0)),
            pl.BlockSpec((K_SHARD, N_GLOBAL), lambda t: (0, 0)),
        ],
        out_specs=pl.BlockSpec((M_CHUNK, STEP_N), lambda t: (0, t)),
        out_shape=jax.ShapeDtypeStruct((M_CHUNK, N_GLOBAL), jnp.bfloat16),
        scratch_shapes=[
            pltpu.VMEM((N_DEV, M_CHUNK, TILE_N), jnp.bfloat16),
            pltpu.VMEM((N_DEV, M_CHUNK, TILE_N), jnp.bfloat16),
            pltpu.SemaphoreType.DMA((N_DEV - 1, 2)),
            pltpu.SemaphoreType.DMA((N_DEV - 1, 2)),
            pltpu.SemaphoreType.DMA((N_DEV - 1, 2)),
            pltpu.SemaphoreType.DMA((N_DEV - 1, 2)),
        ],
        compiler_params=pltpu.CompilerParams(
            collective_id=0,
            dimension_semantics=("arbitrary",),
            vmem_limit_bytes=60 * 1024 * 1024,
        ),
    )(x.astype(jnp.bfloat16), w_mat.astype(jnp.bfloat16))


# device time: 342434 ns/iter; 1.1410x vs baseline; 1.0173x over previous
import jax
import jax.numpy as jnp
from jax import lax
from jax.experimental import pallas as pl
from jax.experimental.pallas import tpu as pltpu

N_DEV = 4
M = 4096
K_SHARD = 1024
N_GLOBAL = 8192
M_CHUNK = M // N_DEV
TILE_N = 1024
STEP_N = 2 * TILE_N
N_STEPS = N_GLOBAL // STEP_N


def kernel(x, w_mat):
    def body(x_ref, w_ref, o_ref, comm_a, comm_b,
             send_a, recv_a, send_b, recv_b):
        t = pl.program_id(0)
        d = lax.axis_index("i")
        left = lax.rem(d + N_DEV - 1, N_DEV)
        right = lax.rem(d + 1, N_DEV)
        col_a = t * STEP_N
        col_b = col_a + TILE_N

        barrier = pltpu.get_barrier_semaphore()
        pl.semaphore_signal(
            barrier, inc=1, device_id=(left,),
            device_id_type=pl.DeviceIdType.MESH,
        )
        pl.semaphore_signal(
            barrier, inc=1, device_id=(right,),
            device_id_type=pl.DeviceIdType.MESH,
        )
        pl.semaphore_wait(barrier, 2)

        def part(c, col):
            return jnp.dot(
                x_ref[pl.ds(c * M_CHUNK, M_CHUNK), :],
                w_ref[:, pl.ds(col, TILE_N)],
                preferred_element_type=jnp.float32,
            )

        @pl.when(t == 0)
        def _():
            comm_a[0, :, :] = part(lax.rem(d + N_DEV - 1, N_DEV),
                                   col_a).astype(jnp.bfloat16)
            comm_b[0, :, :] = part(lax.rem(d + 1, N_DEV),
                                   col_b).astype(jnp.bfloat16)

        HALF = M_CHUNK // 2

        def hop_rdma(comm, sems_s, sems_r, s, h, nbr):
            lo = h * HALF
            return pltpu.make_async_remote_copy(
                src_ref=comm.at[s, lo:lo + HALF],
                dst_ref=comm.at[s + 1, lo:lo + HALF],
                send_sem=sems_s.at[s, h],
                recv_sem=sems_r.at[s, h],
                device_id=(nbr,),
                device_id_type=pl.DeviceIdType.MESH,
            )

        def add_half(comm, p, s, h):
            lo = h * HALF
            comm[s + 1, lo:lo + HALF, :] = (
                comm[s + 1, lo:lo + HALF, :].astype(jnp.float32)
                + p[lo:lo + HALF, :].astype(jnp.float32)
            ).astype(jnp.bfloat16)

        def silu_out_half(comm, p, h, col0):
            lo = h * HALF
            y = (
                comm[N_DEV - 1, lo:lo + HALF, :].astype(jnp.float32)
                + p[lo:lo + HALF, :].astype(jnp.float32)
            )
            o_ref[lo:lo + HALF, col0:col0 + TILE_N] = (
                y * jax.nn.sigmoid(y)
            ).astype(jnp.bfloat16)

        hop_rdma(comm_a, send_a, recv_a, 0, 0, right).start()
        hop_rdma(comm_b, send_b, recv_b, 0, 0, left).start()
        hop_rdma(comm_a, send_a, recv_a, 0, 1, right).start()
        hop_rdma(comm_b, send_b, recv_b, 0, 1, left).start()
        pa = part(lax.rem(d + 2 * N_DEV - 2, N_DEV),
                  col_a).astype(jnp.bfloat16)
        pb = part(lax.rem(d + 2, N_DEV), col_b).astype(jnp.bfloat16)
        for s in range(N_DEV - 1):
            last = s == N_DEV - 2
            hop_rdma(comm_a, send_a, recv_a, s, 0, right).wait()
            if last:
                silu_out_half(comm_a, pa, 0, 0)
            else:
                add_half(comm_a, pa, s, 0)
                hop_rdma(comm_a, send_a, recv_a, s + 1, 0, right).start()
            hop_rdma(comm_b, send_b, recv_b, s, 0, left).wait()
            if last:
                silu_out_half(comm_b, pb, 0, TILE_N)
            else:
                add_half(comm_b, pb, s, 0)
                hop_rdma(comm_b, send_b, recv_b, s + 1, 0, left).start()
            hop_rdma(comm_a, send_a, recv_a, s, 1, right).wait()
            if last:
                silu_out_half(comm_a, pa, 1, 0)
            else:
                add_half(comm_a, pa, s, 1)
                hop_rdma(comm_a, send_a, recv_a, s + 1, 1, right).start()
            hop_rdma(comm_b, send_b, recv_b, s, 1, left).wait()
            if last:
                silu_out_half(comm_b, pb, 1, TILE_N)
            else:
                add_half(comm_b, pb, s, 1)
                hop_rdma(comm_b, send_b, recv_b, s + 1, 1, left).start()
                ca = lax.rem(d + 2 * N_DEV - 3 - s, N_DEV)
                cb = lax.rem(d + 3 + s, N_DEV)
                pa = part(ca, col_a).astype(jnp.bfloat16)
                pb = part(cb, col_b).astype(jnp.bfloat16)
            if s == 1:
                @pl.when(t + 1 < N_STEPS)
                def _():
                    comm_a[0, :, :] = part(
                        lax.rem(d + N_DEV - 1, N_DEV),
                        col_a + STEP_N).astype(jnp.bfloat16)
                    comm_b[0, :, :] = part(
                        lax.rem(d + 1, N_DEV),
                        col_b + STEP_N).astype(jnp.bfloat16)

    return pl.pallas_call(
        body,
        grid=(N_STEPS,),
        in_specs=[
            pl.BlockSpec((M, K_SHARD), lambda t: (0, 0)),
            pl.BlockSpec((K_SHARD, N_GLOBAL), lambda t: (0, 0)),
        ],
        out_specs=pl.BlockSpec((M_CHUNK, STEP_N), lambda t: (0, t)),
        out_shape=jax.ShapeDtypeStruct((M_CHUNK, N_GLOBAL), jnp.bfloat16),
        scratch_shapes=[
            pltpu.VMEM((N_DEV, M_CHUNK, TILE_N), jnp.bfloat16),
            pltpu.VMEM((N_DEV, M_CHUNK, TILE_N), jnp.bfloat16),
            pltpu.SemaphoreType.DMA((N_DEV - 1, 2)),
            pltpu.SemaphoreType.DMA((N_DEV - 1, 2)),
            pltpu.SemaphoreType.DMA((N_DEV - 1, 2)),
            pltpu.SemaphoreType.DMA((N_DEV - 1, 2)),
        ],
        compiler_params=pltpu.CompilerParams(
            collective_id=0,
            dimension_semantics=("arbitrary",),
            vmem_limit_bytes=60 * 1024 * 1024,
        ),
    )(x.astype(jnp.bfloat16), w_mat.astype(jnp.bfloat16))


# device time: 340047 ns/iter; 1.1490x vs baseline; 1.0070x over previous
import jax
import jax.numpy as jnp
from jax import lax
from jax.experimental import pallas as pl
from jax.experimental.pallas import tpu as pltpu

N_DEV = 4
M = 4096
K_SHARD = 1024
N_GLOBAL = 8192
M_CHUNK = M // N_DEV
TILE_N = 1024
STEP_N = 2 * TILE_N
N_STEPS = N_GLOBAL // STEP_N


def kernel(x, w_mat):
    def body(x_ref, w_ref, o_ref, comm_a, comm_b,
             send_a, recv_a, send_b, recv_b):
        t = pl.program_id(0)
        d = lax.axis_index("i")
        left = lax.rem(d + N_DEV - 1, N_DEV)
        right = lax.rem(d + 1, N_DEV)
        col_a = t * STEP_N
        col_b = col_a + TILE_N

        @pl.when(t == 0)
        def _():
            barrier = pltpu.get_barrier_semaphore()
            pl.semaphore_signal(
                barrier, inc=1, device_id=(left,),
                device_id_type=pl.DeviceIdType.MESH,
            )
            pl.semaphore_signal(
                barrier, inc=1, device_id=(right,),
                device_id_type=pl.DeviceIdType.MESH,
            )
            pl.semaphore_wait(barrier, 2)

        def part(c, col):
            return jnp.dot(
                x_ref[pl.ds(c * M_CHUNK, M_CHUNK), :],
                w_ref[:, pl.ds(col, TILE_N)],
                preferred_element_type=jnp.float32,
            )

        @pl.when(t == 0)
        def _():
            comm_a[0, :, :] = part(lax.rem(d + N_DEV - 1, N_DEV),
                                   col_a).astype(jnp.bfloat16)
            comm_b[0, :, :] = part(lax.rem(d + 1, N_DEV),
                                   col_b).astype(jnp.bfloat16)

        HALF = M_CHUNK // 2

        def hop_rdma(comm, sems_s, sems_r, s, h, nbr):
            lo = h * HALF
            return pltpu.make_async_remote_copy(
                src_ref=comm.at[s, lo:lo + HALF],
                dst_ref=comm.at[s + 1, lo:lo + HALF],
                send_sem=sems_s.at[s, h],
                recv_sem=sems_r.at[s, h],
                device_id=(nbr,),
                device_id_type=pl.DeviceIdType.MESH,
            )

        def add_half(comm, p, s, h):
            lo = h * HALF
            comm[s + 1, lo:lo + HALF, :] = (
                comm[s + 1, lo:lo + HALF, :].astype(jnp.float32)
                + p[lo:lo + HALF, :].astype(jnp.float32)
            ).astype(jnp.bfloat16)

        def silu_out_half(comm, p, h, col0):
            lo = h * HALF
            y = (
                comm[N_DEV - 1, lo:lo + HALF, :].astype(jnp.float32)
                + p[lo:lo + HALF, :].astype(jnp.float32)
            )
            o_ref[lo:lo + HALF, col0:col0 + TILE_N] = (
                y * jax.nn.sigmoid(y)
            ).astype(jnp.bfloat16)

        hop_rdma(comm_a, send_a, recv_a, 0, 0, right).start()
        hop_rdma(comm_b, send_b, recv_b, 0, 0, left).start()
        hop_rdma(comm_a, send_a, recv_a, 0, 1, right).start()
        hop_rdma(comm_b, send_b, recv_b, 0, 1, left).start()
        pa = part(lax.rem(d + 2 * N_DEV - 2, N_DEV),
                  col_a).astype(jnp.bfloat16)
        pb = part(lax.rem(d + 2, N_DEV), col_b).astype(jnp.bfloat16)
        for s in range(N_DEV - 1):
            last = s == N_DEV - 2
            hop_rdma(comm_a, send_a, recv_a, s, 0, right).wait()
            hop_rdma(comm_b, send_b, recv_b, s, 0, left).wait()
            if last:
                silu_out_half(comm_a, pa, 0, 0)
                silu_out_half(comm_b, pb, 0, TILE_N)
            else:
                add_half(comm_a, pa, s, 0)
                add_half(comm_b, pb, s, 0)
                hop_rdma(comm_a, send_a, recv_a, s + 1, 0, right).start()
                hop_rdma(comm_b, send_b, recv_b, s + 1, 0, left).start()
            hop_rdma(comm_a, send_a, recv_a, s, 1, right).wait()
            hop_rdma(comm_b, send_b, recv_b, s, 1, left).wait()
            if last:
                silu_out_half(comm_a, pa, 1, 0)
                silu_out_half(comm_b, pb, 1, TILE_N)
            else:
                add_half(comm_a, pa, s, 1)
                add_half(comm_b, pb, s, 1)
                hop_rdma(comm_a, send_a, recv_a, s + 1, 1, right).start()
                hop_rdma(comm_b, send_b, recv_b, s + 1, 1, left).start()
                ca = lax.rem(d + 2 * N_DEV - 3 - s, N_DEV)
                cb = lax.rem(d + 3 + s, N_DEV)
                pa = part(ca, col_a).astype(jnp.bfloat16)
                pb = part(cb, col_b).astype(jnp.bfloat16)
            if s == 1:
                @pl.when(t + 1 < N_STEPS)
                def _():
                    comm_a[0, :, :] = part(
                        lax.rem(d + N_DEV - 1, N_DEV),
                        col_a + STEP_N).astype(jnp.bfloat16)
                    comm_b[0, :, :] = part(
                        lax.rem(d + 1, N_DEV),
                        col_b + STEP_N).astype(jnp.bfloat16)

    return pl.pallas_call(
        body,
        grid=(N_STEPS,),
        in_specs=[
            pl.BlockSpec((M, K_SHARD), lambda t: (0, 0)),
            pl.BlockSpec((K_SHARD, N_GLOBAL), lambda t: (0, 0)),
        ],
        out_specs=pl.BlockSpec((M_CHUNK, STEP_N), lambda t: (0, t)),
        out_shape=jax.ShapeDtypeStruct((M_CHUNK, N_GLOBAL), jnp.bfloat16),
        scratch_shapes=[
            pltpu.VMEM((N_DEV, M_CHUNK, TILE_N), jnp.bfloat16),
            pltpu.VMEM((N_DEV, M_CHUNK, TILE_N), jnp.bfloat16),
            pltpu.SemaphoreType.DMA((N_DEV - 1, 2)),
            pltpu.SemaphoreType.DMA((N_DEV - 1, 2)),
            pltpu.SemaphoreType.DMA((N_DEV - 1, 2)),
            pltpu.SemaphoreType.DMA((N_DEV - 1, 2)),
        ],
        compiler_params=pltpu.CompilerParams(
            collective_id=0,
            dimension_semantics=("arbitrary",),
            vmem_limit_bytes=60 * 1024 * 1024,
        ),
    )(x.astype(jnp.bfloat16), w_mat.astype(jnp.bfloat16))


# device time: 331874 ns/iter; 1.1773x vs baseline; 1.0246x over previous
import jax
import jax.numpy as jnp
from jax import lax
from jax.experimental import pallas as pl
from jax.experimental.pallas import tpu as pltpu

N_DEV = 4
M = 4096
K_SHARD = 1024
N_GLOBAL = 8192
M_CHUNK = M // N_DEV
HALF = M_CHUNK // 2
TILE_N = 1024
STEP_N = 2 * TILE_N
N_STEPS = N_GLOBAL // STEP_N


def kernel(x, w_mat):
    def body(x_ref, w_ref, o_ref, comm_a, comm_b, stage,
             send_a, recv_a, send_b, recv_b, stage_sem):
        d = lax.axis_index("i")
        left = lax.rem(d + N_DEV - 1, N_DEV)
        right = lax.rem(d + 1, N_DEV)

        barrier = pltpu.get_barrier_semaphore()
        pl.semaphore_signal(
            barrier, inc=1, device_id=(left,),
            device_id_type=pl.DeviceIdType.MESH,
        )
        pl.semaphore_signal(
            barrier, inc=1, device_id=(right,),
            device_id_type=pl.DeviceIdType.MESH,
        )
        pl.semaphore_wait(barrier, 2)

        def part(c, col):
            return jnp.dot(
                x_ref[pl.ds(c * M_CHUNK, M_CHUNK), :],
                w_ref[:, col:col + TILE_N],
                preferred_element_type=jnp.float32,
            )

        def hop_rdma(comm, sems_s, sems_r, s, h, nbr):
            lo = h * HALF
            return pltpu.make_async_remote_copy(
                src_ref=comm.at[s, lo:lo + HALF],
                dst_ref=comm.at[s + 1, lo:lo + HALF],
                send_sem=sems_s.at[s, h],
                recv_sem=sems_r.at[s, h],
                device_id=(nbr,),
                device_id_type=pl.DeviceIdType.MESH,
            )

        def start_hop(s, h):
            hop_rdma(comm_a, send_a, recv_a, s, h, right).start()
            hop_rdma(comm_b, send_b, recv_b, s, h, left).start()

        def wait_hop(s, h):
            hop_rdma(comm_a, send_a, recv_a, s, h, right).wait()
            hop_rdma(comm_b, send_b, recv_b, s, h, left).wait()

        def add_half(comm, p, s, h):
            lo = h * HALF
            comm[s + 1, lo:lo + HALF, :] = (
                comm[s + 1, lo:lo + HALF, :].astype(jnp.float32)
                + p[lo:lo + HALF, :].astype(jnp.float32)
            ).astype(jnp.bfloat16)

        def silu_half(comm, p, h, col0):
            lo = h * HALF
            y = (
                comm[N_DEV - 1, lo:lo + HALF, :].astype(jnp.float32)
                + p[lo:lo + HALF, :].astype(jnp.float32)
            )
            stage[lo:lo + HALF, col0:col0 + TILE_N] = (
                y * jax.nn.sigmoid(y)
            ).astype(jnp.bfloat16)

        def stage_dma(col):
            return pltpu.make_async_copy(
                stage, o_ref.at[:, col:col + STEP_N], stage_sem.at[0],
            )

        ca0 = lax.rem(d + N_DEV - 1, N_DEV)
        cb0 = lax.rem(d + 1, N_DEV)

        comm_a[0, :, :] = part(ca0, 0).astype(jnp.bfloat16)
        comm_b[0, :, :] = part(cb0, TILE_N).astype(jnp.bfloat16)
        start_hop(0, 0)
        start_hop(0, 1)
        pa = part(lax.rem(d + 2 * N_DEV - 2, N_DEV), 0).astype(jnp.bfloat16)
        pb = part(lax.rem(d + 2, N_DEV), TILE_N).astype(jnp.bfloat16)

        for st in range(N_STEPS):
            col_a = st * STEP_N
            col_b = col_a + TILE_N
            last_st = st == N_STEPS - 1
            for s in range(N_DEV - 2):
                wait_hop(s, 0)
                add_half(comm_a, pa, s, 0)
                add_half(comm_b, pb, s, 0)
                start_hop(s + 1, 0)
                wait_hop(s, 1)
                add_half(comm_a, pa, s, 1)
                add_half(comm_b, pb, s, 1)
                start_hop(s + 1, 1)
                ca = lax.rem(d + 2 * N_DEV - 3 - s, N_DEV)
                cb = lax.rem(d + 3 + s, N_DEV)
                pa = part(ca, col_a).astype(jnp.bfloat16)
                pb = part(cb, col_b).astype(jnp.bfloat16)
                if s == 1 and not last_st:
                    comm_a[0, :, :] = part(
                        ca0, col_a + STEP_N).astype(jnp.bfloat16)
                    comm_b[0, :, :] = part(
                        cb0, col_b + STEP_N).astype(jnp.bfloat16)
            wait_hop(N_DEV - 2, 0)
            if not last_st:
                start_hop(0, 0)
            if st > 0:
                stage_dma(col_a - STEP_N).wait()
            silu_half(comm_a, pa, 0, 0)
            silu_half(comm_b, pb, 0, TILE_N)
            wait_hop(N_DEV - 2, 1)
            if not last_st:
                start_hop(0, 1)
            silu_half(comm_a, pa, 1, 0)
            silu_half(comm_b, pb, 1, TILE_N)
            stage_dma(col_a).start()
            if not last_st:
                pa = part(lax.rem(d + 2 * N_DEV - 2, N_DEV),
                          col_a + STEP_N).astype(jnp.bfloat16)
                pb = part(lax.rem(d + 2, N_DEV),
                          col_b + STEP_N).astype(jnp.bfloat16)
            else:
                stage_dma(col_a).wait()

    return pl.pallas_call(
        body,
        in_specs=[
            pl.BlockSpec(memory_space=pltpu.MemorySpace.VMEM),
            pl.BlockSpec(memory_space=pltpu.MemorySpace.VMEM),
        ],
        out_specs=pl.BlockSpec(memory_space=pl.ANY),
        out_shape=jax.ShapeDtypeStruct((M_CHUNK, N_GLOBAL), jnp.bfloat16),
        scratch_shapes=[
            pltpu.VMEM((N_DEV, M_CHUNK, TILE_N), jnp.bfloat16),
            pltpu.VMEM((N_DEV, M_CHUNK, TILE_N), jnp.bfloat16),
            pltpu.VMEM((M_CHUNK, STEP_N), jnp.bfloat16),
            pltpu.SemaphoreType.DMA((N_DEV - 1, 2)),
            pltpu.SemaphoreType.DMA((N_DEV - 1, 2)),
            pltpu.SemaphoreType.DMA((N_DEV - 1, 2)),
            pltpu.SemaphoreType.DMA((N_DEV - 1, 2)),
            pltpu.SemaphoreType.DMA((1,)),
        ],
        compiler_params=pltpu.CompilerParams(
            collective_id=0,
            vmem_limit_bytes=60 * 1024 * 1024,
        ),
    )(x.astype(jnp.bfloat16), w_mat.astype(jnp.bfloat16))


# device time: 324387 ns/iter; 1.2045x vs baseline; 1.0231x over previous
import jax
import jax.numpy as jnp
from jax import lax
from jax.experimental import pallas as pl
from jax.experimental.pallas import tpu as pltpu

N_DEV = 4
M = 4096
K_SHARD = 1024
N_GLOBAL = 8192
M_CHUNK = M // N_DEV
HALF = M_CHUNK // 2
TILE_N = 1024
STEP_N = 2 * TILE_N
N_STEPS = N_GLOBAL // STEP_N


def kernel(x, w_mat):
    def body(x_ref, w_ref, o_ref, comm_a, comm_b, stage,
             send_a, recv_a, send_b, recv_b, stage_sem):
        d = lax.axis_index("i")
        left = lax.rem(d + N_DEV - 1, N_DEV)
        right = lax.rem(d + 1, N_DEV)

        barrier = pltpu.get_barrier_semaphore()
        pl.semaphore_signal(
            barrier, inc=1, device_id=(left,),
            device_id_type=pl.DeviceIdType.MESH,
        )
        pl.semaphore_signal(
            barrier, inc=1, device_id=(right,),
            device_id_type=pl.DeviceIdType.MESH,
        )
        pl.semaphore_wait(barrier, 2)

        def part(c, col):
            return jnp.dot(
                x_ref[pl.ds(c * M_CHUNK, M_CHUNK), :].astype(jnp.bfloat16),
                w_ref[:, col:col + TILE_N],
                preferred_element_type=jnp.float32,
            )

        def hop_rdma(comm, sems_s, sems_r, s, h, nbr):
            lo = h * HALF
            return pltpu.make_async_remote_copy(
                src_ref=comm.at[s, lo:lo + HALF],
                dst_ref=comm.at[s + 1, lo:lo + HALF],
                send_sem=sems_s.at[s, h],
                recv_sem=sems_r.at[s, h],
                device_id=(nbr,),
                device_id_type=pl.DeviceIdType.MESH,
            )

        def start_hop(s, h):
            hop_rdma(comm_a, send_a, recv_a, s, h, right).start()
            hop_rdma(comm_b, send_b, recv_b, s, h, left).start()

        def wait_hop(s, h):
            hop_rdma(comm_a, send_a, recv_a, s, h, right).wait()
            hop_rdma(comm_b, send_b, recv_b, s, h, left).wait()

        def add_half(comm, p, s, h):
            lo = h * HALF
            comm[s + 1, lo:lo + HALF, :] = (
                comm[s + 1, lo:lo + HALF, :].astype(jnp.float32)
                + p[lo:lo + HALF, :].astype(jnp.float32)
            ).astype(jnp.bfloat16)

        def silu_half(comm, p, h, col0):
            lo = h * HALF
            y = (
                comm[N_DEV - 1, lo:lo + HALF, :].astype(jnp.float32)
                + p[lo:lo + HALF, :].astype(jnp.float32)
            )
            stage[lo:lo + HALF, col0:col0 + TILE_N] = (
                y * jax.nn.sigmoid(y)
            ).astype(jnp.bfloat16)

        def stage_dma(col):
            return pltpu.make_async_copy(
                stage, o_ref.at[:, col:col + STEP_N], stage_sem.at[0],
            )

        ca0 = lax.rem(d + N_DEV - 1, N_DEV)
        cb0 = lax.rem(d + 1, N_DEV)

        comm_a[0, :, :] = part(ca0, 0).astype(jnp.bfloat16)
        comm_b[0, :, :] = part(cb0, TILE_N).astype(jnp.bfloat16)
        start_hop(0, 0)
        start_hop(0, 1)
        pa = part(lax.rem(d + 2 * N_DEV - 2, N_DEV), 0).astype(jnp.bfloat16)
        pb = part(lax.rem(d + 2, N_DEV), TILE_N).astype(jnp.bfloat16)

        for st in range(N_STEPS):
            col_a = st * STEP_N
            col_b = col_a + TILE_N
            last_st = st == N_STEPS - 1
            for s in range(N_DEV - 2):
                wait_hop(s, 0)
                add_half(comm_a, pa, s, 0)
                add_half(comm_b, pb, s, 0)
                start_hop(s + 1, 0)
                wait_hop(s, 1)
                add_half(comm_a, pa, s, 1)
                add_half(comm_b, pb, s, 1)
                start_hop(s + 1, 1)
                ca = lax.rem(d + 2 * N_DEV - 3 - s, N_DEV)
                cb = lax.rem(d + 3 + s, N_DEV)
                pa = part(ca, col_a).astype(jnp.bfloat16)
                pb = part(cb, col_b).astype(jnp.bfloat16)
                if s == 1 and not last_st:
                    comm_a[0, :, :] = part(
                        ca0, col_a + STEP_N).astype(jnp.bfloat16)
                    comm_b[0, :, :] = part(
                        cb0, col_b + STEP_N).astype(jnp.bfloat16)
            wait_hop(N_DEV - 2, 0)
            if not last_st:
                start_hop(0, 0)
            if st > 0:
                stage_dma(col_a - STEP_N).wait()
            silu_half(comm_a, pa, 0, 0)
            silu_half(comm_b, pb, 0, TILE_N)
            wait_hop(N_DEV - 2, 1)
            if not last_st:
                start_hop(0, 1)
            silu_half(comm_a, pa, 1, 0)
            silu_half(comm_b, pb, 1, TILE_N)
            stage_dma(col_a).start()
            if not last_st:
                pa = part(lax.rem(d + 2 * N_DEV - 2, N_DEV),
                          col_a + STEP_N).astype(jnp.bfloat16)
                pb = part(lax.rem(d + 2, N_DEV),
                          col_b + STEP_N).astype(jnp.bfloat16)
            else:
                stage_dma(col_a).wait()

    return pl.pallas_call(
        body,
        in_specs=[
            pl.BlockSpec(memory_space=pltpu.MemorySpace.VMEM),
            pl.BlockSpec(memory_space=pltpu.MemorySpace.VMEM),
        ],
        out_specs=pl.BlockSpec(memory_space=pl.ANY),
        out_shape=jax.ShapeDtypeStruct((M_CHUNK, N_GLOBAL), jnp.bfloat16),
        scratch_shapes=[
            pltpu.VMEM((N_DEV, M_CHUNK, TILE_N), jnp.bfloat16),
            pltpu.VMEM((N_DEV, M_CHUNK, TILE_N), jnp.bfloat16),
            pltpu.VMEM((M_CHUNK, STEP_N), jnp.bfloat16),
            pltpu.SemaphoreType.DMA((N_DEV - 1, 2)),
            pltpu.SemaphoreType.DMA((N_DEV - 1, 2)),
            pltpu.SemaphoreType.DMA((N_DEV - 1, 2)),
            pltpu.SemaphoreType.DMA((N_DEV - 1, 2)),
            pltpu.SemaphoreType.DMA((1,)),
        ],
        compiler_params=pltpu.CompilerParams(
            collective_id=0,
            vmem_limit_bytes=62 * 1024 * 1024,
        ),
    )(x, w_mat.astype(jnp.bfloat16))
